# speedup vs baseline: 19.9787x; 19.9787x over previous
import functools

import jax
import jax.numpy as jnp
from jax.experimental import pallas as pl
from jax.experimental.pallas import tpu as pltpu

_B = 2048
_N = 64
_H = 128
_FEAT = 512
_BB = 128


def _tc_body(feat_ref, xi_ref, xt_ref, wu_ref, bu_ref, wli_ref, wlt_ref,
             wri_ref, wrt_ref, bli_ref, blt_ref, out_ref):
    inv_n = jnp.float32(1.0 / _N)
    agg_i = jnp.sum(xi_ref[...], axis=1) * inv_n
    agg_t = jnp.sum(xt_ref[...], axis=1) * inv_n
    dn = (((1,), (1,)), ((), ()))
    user = jax.lax.dot_general(feat_ref[...], wu_ref[...], dn,
                               preferred_element_type=jnp.float32)
    user = user + bu_ref[...]
    acc = jax.lax.dot_general(agg_i, wli_ref[...], dn,
                              preferred_element_type=jnp.float32)
    acc = acc + jax.lax.dot_general(agg_t, wlt_ref[...], dn,
                                    preferred_element_type=jnp.float32)
    wr = wri_ref[...] + wrt_ref[...]
    acc = acc + jax.lax.dot_general(user, wr, dn,
                                    preferred_element_type=jnp.float32)
    acc = acc + bli_ref[...] + blt_ref[...]
    out_ref[...] = jnp.maximum(acc, 0.0)


@jax.jit
def kernel(input_text, input_img, base_text_features, base_img_features,
           W_user, b_user, Wl_img, bl_img, Wr_img, Wl_txt, bl_txt, Wr_txt):
    feat = jnp.concatenate([input_text[:, 0, :], input_img[:, 0, :]], axis=1)
    grid = (_B // _BB,)
    full = lambda shape: pl.BlockSpec(shape, lambda i: (0,) * len(shape))
    out = pl.pallas_call(
        _tc_body,
        grid=grid,
        in_specs=[
            pl.BlockSpec((_BB, _FEAT), lambda i: (i, 0)),
            pl.BlockSpec((_BB, _N, _H), lambda i: (i, 0, 0)),
            pl.BlockSpec((_BB, _N, _H), lambda i: (i, 0, 0)),
            full((_H, _FEAT)),
            full((1, _H)),
            full((_H, _H)),
            full((_H, _H)),
            full((_H, _H)),
            full((_H, _H)),
            full((1, _H)),
            full((1, _H)),
        ],
        out_specs=pl.BlockSpec((_BB, _H), lambda i: (i, 0)),
        out_shape=jax.ShapeDtypeStruct((_B, _H), jnp.float32),
    )(feat, base_img_features, base_text_features,
      W_user, b_user.reshape(1, _H), Wl_img, Wl_txt, Wr_img, Wr_txt,
      bl_img.reshape(1, _H), bl_txt.reshape(1, _H))
    return out
